# Initial kernel scaffold; baseline (speedup 1.0000x reference)
import jax
import jax.numpy as jnp
from jax.experimental import pallas as pl


def kernel(x, Wg, We, Wv, Wo):
    raise NotImplementedError("write your pallas kernel here")


# speedup vs baseline: 2.4076x; 2.4076x over previous
import functools

import jax
import jax.numpy as jnp
from jax.experimental import pallas as pl

D_MODEL = 1024
D_FF = 4096
N_EXP = 8
SEQ = 2048
N_BLK = 512


def _router_kernel(x_ref, wg_ref, probs_ref, w_ref):
    x = x_ref[...]
    logits = jax.lax.dot_general(
        x, wg_ref[...], (((1,), (1,)), ((), ())),
        preferred_element_type=jnp.float32)
    m = jnp.max(logits, axis=-1, keepdims=True)
    ex = jnp.exp(logits - m)
    probs = ex / jnp.sum(ex, axis=-1, keepdims=True)
    probs_ref[...] = probs
    idx = jax.lax.broadcasted_iota(jnp.int32, probs.shape, 1)
    m1 = jnp.max(probs, axis=-1, keepdims=True)
    i1 = jnp.min(jnp.where(probs == m1, idx, N_EXP), axis=-1, keepdims=True)
    mask1 = idx == i1
    p2 = jnp.where(mask1, -1.0, probs)
    m2 = jnp.max(p2, axis=-1, keepdims=True)
    i2 = jnp.min(jnp.where(p2 == m2, idx, N_EXP), axis=-1, keepdims=True)
    mask2 = idx == i2
    denom = m1 + m2
    w_ref[...] = jnp.where(mask1 | mask2, probs, 0.0) / denom


def _ffn_kernel(x_ref, w_ref, we_ref, wv_ref, wo_ref, out_ref):
    e = pl.program_id(0)
    n = pl.program_id(1)

    @pl.when((e == 0) & (n == 0))
    def _init():
        out_ref[...] = jnp.zeros_like(out_ref)

    x = x_ref[...]
    g = jax.lax.dot_general(x, we_ref[0], (((1,), (1,)), ((), ())),
                            preferred_element_type=jnp.float32)
    v = jax.lax.dot_general(x, wv_ref[0], (((1,), (1,)), ((), ())),
                            preferred_element_type=jnp.float32)
    h = (0.5 * g * (1.0 + jax.lax.erf(g * 0.7071067811865476))) * v
    part = jax.lax.dot_general(h, wo_ref[0], (((1,), (1,)), ((), ())),
                               preferred_element_type=jnp.float32)
    idx = jax.lax.broadcasted_iota(jnp.int32, (SEQ, N_EXP), 1)
    w = w_ref[...]
    wcol = jnp.sum(jnp.where(idx == e, w, 0.0), axis=-1, keepdims=True)
    out_ref[...] += part * wcol


@jax.jit
def kernel(x, Wg, We, Wv, Wo):
    x2 = x.reshape(SEQ, D_MODEL)
    probs, w = pl.pallas_call(
        _router_kernel,
        out_shape=(
            jax.ShapeDtypeStruct((SEQ, N_EXP), jnp.float32),
            jax.ShapeDtypeStruct((SEQ, N_EXP), jnp.float32),
        ),
    )(x2, Wg)
    out = pl.pallas_call(
        _ffn_kernel,
        grid=(N_EXP, D_FF // N_BLK),
        in_specs=[
            pl.BlockSpec((SEQ, D_MODEL), lambda e, n: (0, 0)),
            pl.BlockSpec((SEQ, N_EXP), lambda e, n: (0, 0)),
            pl.BlockSpec((1, N_BLK, D_MODEL), lambda e, n: (e, n, 0)),
            pl.BlockSpec((1, N_BLK, D_MODEL), lambda e, n: (e, n, 0)),
            pl.BlockSpec((1, D_MODEL, N_BLK), lambda e, n: (e, 0, n)),
        ],
        out_specs=pl.BlockSpec((SEQ, D_MODEL), lambda e, n: (0, 0)),
        out_shape=jax.ShapeDtypeStruct((SEQ, D_MODEL), jnp.float32),
    )(x2, w, We, Wv, Wo)
    return out.reshape(1, SEQ, D_MODEL), probs.reshape(1, SEQ, N_EXP)
